# baseline (speedup 1.0000x reference)
import functools

import jax
import jax.numpy as jnp
from jax.experimental import pallas as pl
from jax.experimental.pallas import tpu as pltpu

_K = 20
_TEMP = 0.01


def _block_kernel(feat_ref, sfeat_ref, x_ref, out_ref, sn_ref):
    f = feat_ref[0]
    s = sfeat_ref[0]
    xb = x_ref[0]

    @pl.when(pl.program_id(1) == 0)
    def _():
        sn_ref[...] = jnp.sum(s * s, axis=1, keepdims=True).T

    fn = jnp.sum(f * f, axis=1, keepdims=True)
    sn = sn_ref[...]
    dot = jax.lax.dot_general(
        f, s, (((1,), (1,)), ((), ())),
        preferred_element_type=jnp.float32)
    ds = jnp.maximum(fn + sn - 2.0 * dot, 0.0)
    w = jnp.exp(-_TEMP * ds)

    bn, m = w.shape

    wm = w
    removed = jnp.zeros((bn, 1), jnp.float32)
    thr = jnp.zeros((bn, 1), jnp.float32)
    for j in range(_K):
        mx = jnp.max(wm, axis=1, keepdims=True)
        thr = jnp.where(removed < _K, mx, thr)
        if j < _K - 1:
            eq = wm == mx
            removed = removed + jnp.sum(
                jnp.where(eq, 1.0, 0.0), axis=1, keepdims=True)
            wm = jnp.where(eq, -jnp.inf, wm)

    att = jnp.where(w >= thr, w, 0.0)
    norm = jnp.sqrt(jnp.sum(att * att, axis=1, keepdims=True))
    att = att / jnp.maximum(norm, 1e-12)

    out_ref[0] = jax.lax.dot_general(
        att, xb, (((1,), (0,)), ((), ())),
        preferred_element_type=jnp.float32)


@functools.partial(jax.jit, static_argnames=("bn",))
def _run(x, feat, sfeat, bn):
    b, n, c = feat.shape
    _, m, _ = x.shape
    grid = (b, n // bn)
    return pl.pallas_call(
        _block_kernel,
        grid=grid,
        in_specs=[
            pl.BlockSpec((1, bn, c), lambda bi, ni: (bi, ni, 0)),
            pl.BlockSpec((1, m, c), lambda bi, ni: (bi, 0, 0)),
            pl.BlockSpec((1, m, c), lambda bi, ni: (bi, 0, 0)),
        ],
        out_specs=pl.BlockSpec((1, bn, c), lambda bi, ni: (bi, ni, 0)),
        out_shape=jax.ShapeDtypeStruct((b, n, c), jnp.float32),
        scratch_shapes=[pltpu.VMEM((1, m), jnp.float32)],
        compiler_params=pltpu.CompilerParams(
            dimension_semantics=("parallel", "arbitrary")),
    )(feat, sfeat, x)


def kernel(x, feat_before_pooling, feat_after_pooling):
    n = feat_before_pooling.shape[1]
    bn = 512 if n % 512 == 0 else n
    return _run(x, feat_before_pooling, feat_after_pooling, bn)


# speedup vs baseline: 1.0598x; 1.0598x over previous
import functools

import jax
import jax.numpy as jnp
from jax.experimental import pallas as pl
from jax.experimental.pallas import tpu as pltpu

_K = 20
_TEMP = 0.01


def _block_kernel(feat_ref, sfeat_ref, x_ref, out_ref):
    f = feat_ref[0]
    s = sfeat_ref[0]
    xb = x_ref[0]

    fn = jnp.sum(f * f, axis=1, keepdims=True)
    sn = jnp.sum(s * s, axis=1, keepdims=True).T
    dot = jax.lax.dot_general(
        f, s, (((1,), (1,)), ((), ())),
        preferred_element_type=jnp.float32)
    ds = jnp.maximum(fn + sn - 2.0 * dot, 0.0)
    w = jnp.exp(-_TEMP * ds)

    bn, m = w.shape

    wm = w
    removed = jnp.zeros((bn, 1), jnp.float32)
    thr = jnp.zeros((bn, 1), jnp.float32)
    for j in range(_K):
        mx = jnp.max(wm, axis=1, keepdims=True)
        thr = jnp.where(removed < _K, mx, thr)
        if j < _K - 1:
            eq = wm == mx
            removed = removed + jnp.sum(
                jnp.where(eq, 1.0, 0.0), axis=1, keepdims=True)
            wm = jnp.where(eq, -jnp.inf, wm)

    att = jnp.where(w >= thr, w, 0.0)
    norm = jnp.sqrt(jnp.sum(att * att, axis=1, keepdims=True))
    att = att / jnp.maximum(norm, 1e-12)

    out_ref[0] = jax.lax.dot_general(
        att, xb, (((1,), (0,)), ((), ())),
        preferred_element_type=jnp.float32)


@functools.partial(jax.jit, static_argnames=("bn",))
def _run(x, feat, sfeat, bn):
    b, n, c = feat.shape
    _, m, _ = x.shape
    grid = (b, n // bn)
    return pl.pallas_call(
        _block_kernel,
        grid=grid,
        in_specs=[
            pl.BlockSpec((1, bn, c), lambda bi, ni: (bi, ni, 0)),
            pl.BlockSpec((1, m, c), lambda bi, ni: (bi, 0, 0)),
            pl.BlockSpec((1, m, c), lambda bi, ni: (bi, 0, 0)),
        ],
        out_specs=pl.BlockSpec((1, bn, c), lambda bi, ni: (bi, ni, 0)),
        out_shape=jax.ShapeDtypeStruct((b, n, c), jnp.float32),
        compiler_params=pltpu.CompilerParams(
            dimension_semantics=("parallel", "arbitrary")),
    )(feat, sfeat, x)


def kernel(x, feat_before_pooling, feat_after_pooling):
    n = feat_before_pooling.shape[1]
    bn = 512 if n % 512 == 0 else n
    return _run(x, feat_before_pooling, feat_after_pooling, bn)


# speedup vs baseline: 1.4638x; 1.3812x over previous
import functools

import jax
import jax.numpy as jnp
from jax.experimental import pallas as pl
from jax.experimental.pallas import tpu as pltpu

_K = 20
_TEMP = 0.01


def _block_kernel(feat_ref, sfeat_ref, x_ref, out_ref):
    f = feat_ref[0]
    s = sfeat_ref[0]
    xb = x_ref[0]

    fn = jnp.sum(f * f, axis=1, keepdims=True)
    sn = jnp.sum(s * s, axis=1, keepdims=True).T
    dot = jax.lax.dot_general(
        f, s, (((1,), (1,)), ((), ())),
        preferred_element_type=jnp.float32)
    ds = jnp.maximum(fn + sn - 2.0 * dot, 0.0)
    w = jnp.exp(-_TEMP * ds)

    bn, m = w.shape

    g = 8 if m % 8 == 0 and (m // 8) % 128 == 0 else 1
    pw = m // g
    planes = [w[:, p * pw:(p + 1) * pw] for p in range(g)]
    if g == 8:
        net = [(0, 1), (2, 3), (4, 5), (6, 7),
               (0, 2), (1, 3), (4, 6), (5, 7),
               (1, 2), (5, 6),
               (0, 4), (1, 5), (2, 6), (3, 7),
               (2, 4), (3, 5),
               (1, 2), (3, 4), (5, 6)]
        for i, jj in net:
            hi = jnp.maximum(planes[i], planes[jj])
            lo = jnp.minimum(planes[i], planes[jj])
            planes[i], planes[jj] = hi, lo

    removed = jnp.zeros((bn, 1), jnp.float32)
    thr = jnp.zeros((bn, 1), jnp.float32)
    for j in range(_K):
        mx = jnp.max(planes[0], axis=1, keepdims=True)
        thr = jnp.where(removed < _K, mx, thr)
        if j < _K - 1:
            eq = planes[0] == mx
            removed = removed + jnp.sum(
                jnp.where(eq, 1.0, 0.0), axis=1, keepdims=True)
            for p in range(g - 1):
                planes[p] = jnp.where(eq, planes[p + 1], planes[p])
            planes[g - 1] = jnp.where(eq, -jnp.inf, planes[g - 1])

    att = jnp.where(w >= thr, w, 0.0)
    norm = jnp.sqrt(jnp.sum(att * att, axis=1, keepdims=True))
    att = att / jnp.maximum(norm, 1e-12)

    out_ref[0] = jax.lax.dot_general(
        att, xb, (((1,), (0,)), ((), ())),
        preferred_element_type=jnp.float32)


@functools.partial(jax.jit, static_argnames=("bn",))
def _run(x, feat, sfeat, bn):
    b, n, c = feat.shape
    _, m, _ = x.shape
    grid = (b, n // bn)
    return pl.pallas_call(
        _block_kernel,
        grid=grid,
        in_specs=[
            pl.BlockSpec((1, bn, c), lambda bi, ni: (bi, ni, 0)),
            pl.BlockSpec((1, m, c), lambda bi, ni: (bi, 0, 0)),
            pl.BlockSpec((1, m, c), lambda bi, ni: (bi, 0, 0)),
        ],
        out_specs=pl.BlockSpec((1, bn, c), lambda bi, ni: (bi, ni, 0)),
        out_shape=jax.ShapeDtypeStruct((b, n, c), jnp.float32),
        compiler_params=pltpu.CompilerParams(
            dimension_semantics=("parallel", "arbitrary")),
    )(feat, sfeat, x)


def kernel(x, feat_before_pooling, feat_after_pooling):
    n = feat_before_pooling.shape[1]
    bn = 512 if n % 512 == 0 else n
    return _run(x, feat_before_pooling, feat_after_pooling, bn)


# speedup vs baseline: 1.4750x; 1.0077x over previous
import functools

import jax
import jax.numpy as jnp
from jax.experimental import pallas as pl
from jax.experimental.pallas import tpu as pltpu

_K = 20
_TEMP = 0.01


def _block_kernel(feat_ref, sfeat_ref, x_ref, out_ref):
    f = feat_ref[0]
    s = sfeat_ref[0]
    xb = x_ref[0]

    fn = jnp.sum(f * f, axis=1, keepdims=True)
    sn = jnp.sum(s * s, axis=1, keepdims=True).T
    dot = jax.lax.dot_general(
        f, s, (((1,), (1,)), ((), ())),
        preferred_element_type=jnp.float32)
    ds = jnp.maximum(fn + sn - 2.0 * dot, 0.0)
    w = jnp.exp(-_TEMP * ds)

    bn, m = w.shape

    g = 8 if m % 8 == 0 and (m // 8) % 128 == 0 else 1
    pw = m // g
    planes = [w[:, p * pw:(p + 1) * pw] for p in range(g)]
    if g == 8:
        net = [(0, 1), (2, 3), (4, 5), (6, 7),
               (0, 2), (1, 3), (4, 6), (5, 7),
               (1, 2), (5, 6),
               (0, 4), (1, 5), (2, 6), (3, 7),
               (2, 4), (3, 5),
               (1, 2), (3, 4), (5, 6)]
        for i, jj in net:
            hi = jnp.maximum(planes[i], planes[jj])
            lo = jnp.minimum(planes[i], planes[jj])
            planes[i], planes[jj] = hi, lo

    removed = jnp.zeros((bn, 1), jnp.float32)
    thr = jnp.zeros((bn, 1), jnp.float32)
    for j in range(_K):
        mx = jnp.max(planes[0], axis=1, keepdims=True)
        thr = jnp.where(removed < _K, mx, thr)
        if j < _K - 1:
            eq = planes[0] == mx
            removed = removed + jnp.sum(
                jnp.where(eq, 1.0, 0.0), axis=1, keepdims=True)
            depth = min(g - 1, _K - 2 - j)
            neg = jnp.full_like(planes[0], -jnp.inf)
            for p in range(depth + 1):
                src = planes[p + 1] if p + 1 < g else neg
                planes[p] = jnp.where(eq, src, planes[p])

    att = jnp.where(w >= thr, w, 0.0)
    norm = jnp.sqrt(jnp.sum(att * att, axis=1, keepdims=True))
    att = att / jnp.maximum(norm, 1e-12)

    out_ref[0] = jax.lax.dot_general(
        att, xb, (((1,), (0,)), ((), ())),
        preferred_element_type=jnp.float32)


@functools.partial(jax.jit, static_argnames=("bn",))
def _run(x, feat, sfeat, bn):
    b, n, c = feat.shape
    _, m, _ = x.shape
    grid = (b, n // bn)
    return pl.pallas_call(
        _block_kernel,
        grid=grid,
        in_specs=[
            pl.BlockSpec((1, bn, c), lambda bi, ni: (bi, ni, 0)),
            pl.BlockSpec((1, m, c), lambda bi, ni: (bi, 0, 0)),
            pl.BlockSpec((1, m, c), lambda bi, ni: (bi, 0, 0)),
        ],
        out_specs=pl.BlockSpec((1, bn, c), lambda bi, ni: (bi, ni, 0)),
        out_shape=jax.ShapeDtypeStruct((b, n, c), jnp.float32),
        compiler_params=pltpu.CompilerParams(
            dimension_semantics=("parallel", "arbitrary")),
    )(feat, sfeat, x)


def kernel(x, feat_before_pooling, feat_after_pooling):
    n = feat_before_pooling.shape[1]
    bn = 1024 if n % 1024 == 0 else n
    return _run(x, feat_before_pooling, feat_after_pooling, bn)


# speedup vs baseline: 1.4766x; 1.0011x over previous
import functools

import jax
import jax.numpy as jnp
from jax.experimental import pallas as pl
from jax.experimental.pallas import tpu as pltpu

_K = 20
_TEMP = 0.01


def _block_kernel(feat_ref, sfeat_ref, x_ref, out_ref):
    f = feat_ref[0]
    s = sfeat_ref[0]
    xb = x_ref[0]

    fn = jnp.sum(f * f, axis=1, keepdims=True)
    sn = jnp.sum(s * s, axis=1, keepdims=True).T
    dot = jax.lax.dot_general(
        f, s, (((1,), (1,)), ((), ())),
        preferred_element_type=jnp.float32)
    ds = jnp.maximum(fn + sn - 2.0 * dot, 0.0)
    w = jnp.exp(-_TEMP * ds)

    bn, m = w.shape

    g = 8 if m % 8 == 0 and (m // 8) % 128 == 0 else 1
    pw = m // g
    planes = [w[:, p * pw:(p + 1) * pw] for p in range(g)]
    if g == 8:
        net = [(0, 1), (2, 3), (4, 5), (6, 7),
               (0, 2), (1, 3), (4, 6), (5, 7),
               (1, 2), (5, 6),
               (0, 4), (1, 5), (2, 6), (3, 7),
               (2, 4), (3, 5),
               (1, 2), (3, 4), (5, 6)]
        for i, jj in net:
            hi = jnp.maximum(planes[i], planes[jj])
            lo = jnp.minimum(planes[i], planes[jj])
            planes[i], planes[jj] = hi, lo

    removed = jnp.zeros((bn, 1), jnp.float32)
    thr = jnp.zeros((bn, 1), jnp.float32)

    def count_of(eqm):
        return jnp.sum(jnp.where(eqm, 1.0, 0.0), axis=1, keepdims=True)

    for j in range(_K):
        mx = jnp.max(planes[0], axis=1, keepdims=True)
        thr = jnp.where(removed < _K, mx, thr)
        if j < _K - 1:
            eq = planes[0] == mx
            removed = removed + count_of(eq)
            neg = jnp.full_like(planes[0], -jnp.inf)
            for p in range(g):
                src = planes[p + 1] if p + 1 < g else neg
                planes[p] = jnp.where(eq, src, planes[p])

    att = jnp.where(w >= thr, w, 0.0)
    norm = jnp.sqrt(jnp.sum(att * att, axis=1, keepdims=True))
    att = att / jnp.maximum(norm, 1e-12)

    out_ref[0] = jax.lax.dot_general(
        att, xb, (((1,), (0,)), ((), ())),
        preferred_element_type=jnp.float32)


@functools.partial(jax.jit, static_argnames=("bn",))
def _run(x, feat, sfeat, bn):
    b, n, c = feat.shape
    _, m, _ = x.shape
    grid = (b, n // bn)
    return pl.pallas_call(
        _block_kernel,
        grid=grid,
        in_specs=[
            pl.BlockSpec((1, bn, c), lambda bi, ni: (bi, ni, 0)),
            pl.BlockSpec((1, m, c), lambda bi, ni: (bi, 0, 0)),
            pl.BlockSpec((1, m, c), lambda bi, ni: (bi, 0, 0)),
        ],
        out_specs=pl.BlockSpec((1, bn, c), lambda bi, ni: (bi, ni, 0)),
        out_shape=jax.ShapeDtypeStruct((b, n, c), jnp.float32),
        compiler_params=pltpu.CompilerParams(
            dimension_semantics=("parallel", "arbitrary")),
    )(feat, sfeat, x)


def kernel(x, feat_before_pooling, feat_after_pooling):
    n = feat_before_pooling.shape[1]
    bn = 1024 if n % 1024 == 0 else n
    return _run(x, feat_before_pooling, feat_after_pooling, bn)
